# baseline (speedup 1.0000x reference)
import functools

import jax
import jax.numpy as jnp
from jax import lax
from jax.experimental import pallas as pl
from jax.experimental.pallas import tpu as pltpu
from jax.experimental.pallas import tpu_sc as plsc

S = 7
NB = 2
C = 80
L_COORD = 5.0
L_NOOBJ = 0.5
B = 128
NGT = 24
SS = S * S
F = 5 * NB + C
PB_W = SS * F
NW = 32
IPT = B // NW


def _sqrt_newton(x):
    y = (x + 1.0) * 0.5
    for _ in range(6):
        y = 0.5 * (y + x / y)
    return y


@functools.partial(
    pl.kernel,
    mesh=plsc.VectorSubcoreMesh(core_axis_name="c", subcore_axis_name="s"),
    out_type=jax.ShapeDtypeStruct((NW, 16), jnp.float32),
    scratch_types=[
        pltpu.VMEM((IPT, PB_W), jnp.float32),
        pltpu.VMEM((IPT * NGT * 4,), jnp.float32),
        pltpu.VMEM((IPT * NGT,), jnp.int32),
        pltpu.VMEM((16,), jnp.float32),
    ],
    compiler_params=pltpu.CompilerParams(needs_layout_passes=False),
)
def _sc_loss_kernel(preds_hbm, box_hbm, lbl_hbm,
                    out_hbm, pred_v, box_v, lbl_v, part_v):
    wid = lax.axis_index("s") * 2 + lax.axis_index("c")
    pltpu.sync_copy(preds_hbm.at[pl.ds(wid * IPT, IPT)], pred_v)
    pltpu.sync_copy(box_hbm.at[pl.ds(wid * (IPT * NGT * 4), IPT * NGT * 4)],
                    box_v)
    pltpu.sync_copy(lbl_hbm.at[pl.ds(wid * (IPT * NGT), IPT * NGT)], lbl_v)

    lane = lax.iota(jnp.int32, 16)
    fzero = jnp.zeros((16,), jnp.float32)

    coord_acc = jnp.float32(0.0)
    obj_acc = jnp.float32(0.0)
    confw_acc = jnp.float32(0.0)
    confall_acc = jnp.float32(0.0)
    clsA_acc = jnp.float32(0.0)
    b_acc = jnp.float32(0.0)
    c_acc = jnp.float32(0.0)

    lane0 = lane * 0
    for img in range(IPT):
        img_vec = lane0 + img
        cid = []
        lbls = []
        key = []
        pk = []
        pos = []
        valid = []
        se = []
        psel4 = []
        ibest = []
        for ch in range(2):
            posc = jnp.minimum(lane + 16 * ch, NGT - 1)
            bb = (img * NGT + posc) * 4
            gx1 = plsc.load_gather(box_v, [bb])
            gy1 = plsc.load_gather(box_v, [bb + 1])
            gx2 = plsc.load_gather(box_v, [bb + 2])
            gy2 = plsc.load_gather(box_v, [bb + 3])
            lbl = plsc.load_gather(lbl_v, [img * NGT + posc])
            v = lane < (NGT - 16 * ch)
            gcx = (gx1 + gx2) * 0.5
            gcy = (gy1 + gy2) * 0.5
            gw = gx2 - gx1
            gh = gy2 - gy1
            gi = jnp.clip((gcx * S).astype(jnp.int32), 0, S - 1)
            gj = jnp.clip((gcy * S).astype(jnp.int32), 0, S - 1)
            cidc = gj * S + gi
            gif = gi.astype(jnp.float32)
            gjf = gj.astype(jnp.float32)
            fb = cidc * F
            feats = [[plsc.load_gather(pred_v, [img_vec, fb + (5 * n + f)])
                      for f in range(5)] for n in range(NB)]
            g_area = jnp.maximum(gw, 0.0) * jnp.maximum(gh, 0.0)
            ious = []
            for n in range(NB):
                px, py, pw, ph, _ = feats[n]
                pcx = (px + gif) / S
                pcy = (py + gjf) / S
                pw2 = pw * pw
                ph2 = ph * ph
                px1 = pcx - pw2 * 0.5
                py1 = pcy - ph2 * 0.5
                px2 = pcx + pw2 * 0.5
                py2 = pcy + ph2 * 0.5
                ix1 = jnp.maximum(px1, gx1)
                iy1 = jnp.maximum(py1, gy1)
                ix2 = jnp.minimum(px2, gx2)
                iy2 = jnp.minimum(py2, gy2)
                inter = (jnp.maximum(ix2 - ix1, 0.0)
                         * jnp.maximum(iy2 - iy1, 0.0))
                p_area = (jnp.maximum(px2 - px1, 0.0)
                          * jnp.maximum(py2 - py1, 0.0))
                ious.append(inter / (p_area + g_area - inter + 1e-6))
            bestc = ious[1] > ious[0]
            ibc = jnp.where(bestc, ious[1], ious[0])
            ps = [jnp.where(bestc, feats[1][f], feats[0][f]) for f in range(5)]
            tx = gcx * S - gif
            ty = gcy * S - gjf
            tw = _sqrt_newton(gw)
            th = _sqrt_newton(gh)
            d0 = ps[0] - tx
            d1 = ps[1] - ty
            d2 = ps[2] - tw
            d3 = ps[3] - th
            sec = d0 * d0 + d1 * d1 + d2 * d2 + d3 * d3
            cid.append(cidc)
            lbls.append(lbl)
            key.append(cidc * 2 + bestc.astype(jnp.int32))
            pk.append(cidc * C + lbl)
            pos.append(lane + 16 * ch)
            valid.append(v)
            se.append(sec)
            psel4.append(ps[4])
            ibest.append(ibc)

        bfalse = jnp.zeros((16,), jnp.bool_)
        clob0 = clob1 = dupP0 = dupP1 = dupC0 = dupC1 = bfalse
        for j in range(NGT):
            chj = j // 16
            lm = lane == (j - 16 * chj)
            key_j = jnp.max(jnp.where(lm, key[chj], 0))
            pk_j = jnp.max(jnp.where(lm, pk[chj], 0))
            cid_j = jnp.max(jnp.where(lm, cid[chj], 0))
            clob0 = clob0 | ((key[0] == key_j) & (pos[0] < j))
            clob1 = clob1 | ((key[1] == key_j) & (pos[1] < j))
            dupP0 = dupP0 | ((pk[0] == pk_j) & (pos[0] > j))
            dupP1 = dupP1 | ((pk[1] == pk_j) & (pos[1] > j))
            dupC0 = dupC0 | ((cid[0] == cid_j) & (pos[0] > j))
            dupC1 = dupC1 | ((cid[1] == cid_j) & (pos[1] > j))

        winner = [valid[0] & ~clob0, valid[1] & ~clob1]
        distinct = [valid[0] & ~dupP0, valid[1] & ~dupP1]
        dcell = [valid[0] & ~dupC0, valid[1] & ~dupC1]

        for ch in range(2):
            w = winner[ch]
            coord_acc += jnp.sum(jnp.where(w, se[ch], fzero))
            dob = psel4[ch] - ibest[ch]
            obj_acc += jnp.sum(jnp.where(w, dob * dob, fzero))
            confw_acc += jnp.sum(jnp.where(w, psel4[ch] * psel4[ch], fzero))
            gcls = plsc.load_gather(
                pred_v, [img_vec, cid[ch] * F + (5 * NB) + lbls[ch]])
            b_acc += jnp.sum(jnp.where(distinct[ch], gcls, fzero))
            c_acc += jnp.sum(
                jnp.where(distinct[ch], jnp.ones((16,), jnp.float32), fzero))

        cssv0 = cssv1 = fzero
        fb0 = cid[0] * F + (5 * NB)
        fb1 = cid[1] * F + (5 * NB)
        for l in range(C):
            g0 = plsc.load_gather(pred_v, [img_vec, fb0 + l])
            g1 = plsc.load_gather(pred_v, [img_vec, fb1 + l])
            cssv0 = cssv0 + g0 * g0
            cssv1 = cssv1 + g1 * g1
        clsA_acc += jnp.sum(jnp.where(dcell[0], cssv0, fzero))
        clsA_acc += jnp.sum(jnp.where(dcell[1], cssv1, fzero))

        for cc in range(4):
            cells = lane + 16 * cc
            cmask = cells < SS
            cellc = jnp.minimum(cells, SS - 1)
            v4 = plsc.load_gather(pred_v, [img_vec, cellc * F + 4])
            v9 = plsc.load_gather(pred_v, [img_vec, cellc * F + 9])
            confall_acc += jnp.sum(jnp.where(cmask, v4 * v4 + v9 * v9, fzero))

    p = jnp.zeros((16,), jnp.float32)
    p = jnp.where(lane == 0, coord_acc * L_COORD, p)
    p = jnp.where(lane == 1, obj_acc, p)
    p = jnp.where(lane == 2, confw_acc, p)
    p = jnp.where(lane == 3, confall_acc, p)
    p = jnp.where(lane == 4, clsA_acc, p)
    p = jnp.where(lane == 5, b_acc, p)
    p = jnp.where(lane == 6, c_acc, p)
    part_v[...] = p
    pltpu.sync_copy(part_v, out_hbm.at[wid])


def kernel(preds, boxes, labels):
    preds_flat = preds.reshape(B, PB_W)
    box_flat = boxes.reshape(B * NGT * 4)
    lbl_flat = labels.reshape(B * NGT)
    parts = _sc_loss_kernel(preds_flat, box_flat, lbl_flat)
    coord_loss = jnp.sum(parts[:, 0])
    obj_loss = jnp.sum(parts[:, 1])
    noobj_loss = (jnp.sum(parts[:, 3]) - jnp.sum(parts[:, 2])) * L_NOOBJ
    cls_loss = (jnp.sum(parts[:, 4]) - 2.0 * jnp.sum(parts[:, 5])
                + jnp.sum(parts[:, 6]))
    total = coord_loss + obj_loss + noobj_loss + cls_loss
    return total, coord_loss, obj_loss, noobj_loss, cls_loss


# speedup vs baseline: 1.0715x; 1.0715x over previous
import functools

import jax
import jax.numpy as jnp
from jax import lax
from jax.experimental import pallas as pl
from jax.experimental.pallas import tpu as pltpu
from jax.experimental.pallas import tpu_sc as plsc

S = 7
NB = 2
C = 80
L_COORD = 5.0
L_NOOBJ = 0.5
B = 128
NGT = 24
SS = S * S
F = 5 * NB + C
PB_W = SS * F
NW = 32
IPT = B // NW


def _sqrt_newton(x):
    y = (x + 1.0) * 0.5
    for _ in range(6):
        y = 0.5 * (y + x / y)
    return y


@functools.partial(
    pl.kernel,
    mesh=plsc.VectorSubcoreMesh(core_axis_name="c", subcore_axis_name="s"),
    out_type=jax.ShapeDtypeStruct((NW, 16), jnp.float32),
    scratch_types=[
        pltpu.VMEM((IPT * PB_W,), jnp.float32),
        pltpu.VMEM((IPT * NGT * 4,), jnp.float32),
        pltpu.VMEM((IPT * NGT,), jnp.int32),
        pltpu.VMEM((16,), jnp.float32),
    ],
    compiler_params=pltpu.CompilerParams(needs_layout_passes=False),
)
def _sc_loss_kernel(preds_hbm, box_hbm, lbl_hbm,
                    out_hbm, pred_v, box_v, lbl_v, part_v):
    wid = lax.axis_index("s") * 2 + lax.axis_index("c")
    pltpu.sync_copy(preds_hbm.at[pl.ds(wid * (IPT * PB_W), IPT * PB_W)], pred_v)
    pltpu.sync_copy(box_hbm.at[pl.ds(wid * (IPT * NGT * 4), IPT * NGT * 4)],
                    box_v)
    pltpu.sync_copy(lbl_hbm.at[pl.ds(wid * (IPT * NGT), IPT * NGT)], lbl_v)

    lane = lax.iota(jnp.int32, 16)
    fzero = jnp.zeros((16,), jnp.float32)

    coord_acc = jnp.float32(0.0)
    obj_acc = jnp.float32(0.0)
    confw_acc = jnp.float32(0.0)
    confall_acc = jnp.float32(0.0)
    clsA_acc = jnp.float32(0.0)
    b_acc = jnp.float32(0.0)
    c_acc = jnp.float32(0.0)

    def img_body(img, accs):
        (coord_acc, obj_acc, confw_acc, confall_acc,
         clsA_acc, b_acc, c_acc) = accs
        pbase = img * PB_W
        cid = []
        lbls = []
        key = []
        pk = []
        pos = []
        valid = []
        se = []
        psel4 = []
        ibest = []
        for ch in range(2):
            posc = jnp.minimum(lane + 16 * ch, NGT - 1)
            bb = (img * NGT + posc) * 4
            gx1 = plsc.load_gather(box_v, [bb])
            gy1 = plsc.load_gather(box_v, [bb + 1])
            gx2 = plsc.load_gather(box_v, [bb + 2])
            gy2 = plsc.load_gather(box_v, [bb + 3])
            lbl = plsc.load_gather(lbl_v, [img * NGT + posc])
            v = lane < (NGT - 16 * ch)
            gcx = (gx1 + gx2) * 0.5
            gcy = (gy1 + gy2) * 0.5
            gw = gx2 - gx1
            gh = gy2 - gy1
            gi = jnp.clip((gcx * S).astype(jnp.int32), 0, S - 1)
            gj = jnp.clip((gcy * S).astype(jnp.int32), 0, S - 1)
            cidc = gj * S + gi
            gif = gi.astype(jnp.float32)
            gjf = gj.astype(jnp.float32)
            fb = cidc * F + pbase
            feats = [[plsc.load_gather(pred_v, [fb + (5 * n + f)])
                      for f in range(5)] for n in range(NB)]
            g_area = jnp.maximum(gw, 0.0) * jnp.maximum(gh, 0.0)
            ious = []
            for n in range(NB):
                px, py, pw, ph, _ = feats[n]
                pcx = (px + gif) / S
                pcy = (py + gjf) / S
                pw2 = pw * pw
                ph2 = ph * ph
                px1 = pcx - pw2 * 0.5
                py1 = pcy - ph2 * 0.5
                px2 = pcx + pw2 * 0.5
                py2 = pcy + ph2 * 0.5
                ix1 = jnp.maximum(px1, gx1)
                iy1 = jnp.maximum(py1, gy1)
                ix2 = jnp.minimum(px2, gx2)
                iy2 = jnp.minimum(py2, gy2)
                inter = (jnp.maximum(ix2 - ix1, 0.0)
                         * jnp.maximum(iy2 - iy1, 0.0))
                p_area = (jnp.maximum(px2 - px1, 0.0)
                          * jnp.maximum(py2 - py1, 0.0))
                ious.append(inter / (p_area + g_area - inter + 1e-6))
            bestc = ious[1] > ious[0]
            ibc = jnp.where(bestc, ious[1], ious[0])
            ps = [jnp.where(bestc, feats[1][f], feats[0][f]) for f in range(5)]
            tx = gcx * S - gif
            ty = gcy * S - gjf
            tw = _sqrt_newton(gw)
            th = _sqrt_newton(gh)
            d0 = ps[0] - tx
            d1 = ps[1] - ty
            d2 = ps[2] - tw
            d3 = ps[3] - th
            sec = d0 * d0 + d1 * d1 + d2 * d2 + d3 * d3
            cid.append(cidc)
            lbls.append(lbl)
            key.append(cidc * 2 + bestc.astype(jnp.int32))
            pk.append(cidc * C + lbl)
            pos.append(lane + 16 * ch)
            valid.append(v)
            se.append(sec)
            psel4.append(ps[4])
            ibest.append(ibc)

        bfalse = jnp.zeros((16,), jnp.bool_)

        def pair_body(j, carry):
            clob0, clob1, dupP0, dupP1, dupC0, dupC1 = carry
            jm = jnp.where(j < 16, j, j - 16)
            in0 = j < 16
            lm = lane == jm
            key_j = jnp.max(jnp.where(lm, jnp.where(in0, key[0], key[1]), 0))
            pk_j = jnp.max(jnp.where(lm, jnp.where(in0, pk[0], pk[1]), 0))
            cid_j = jnp.max(jnp.where(lm, jnp.where(in0, cid[0], cid[1]), 0))
            clob0 = clob0 | ((key[0] == key_j) & (pos[0] < j))
            clob1 = clob1 | ((key[1] == key_j) & (pos[1] < j))
            dupP0 = dupP0 | ((pk[0] == pk_j) & (pos[0] > j))
            dupP1 = dupP1 | ((pk[1] == pk_j) & (pos[1] > j))
            dupC0 = dupC0 | ((cid[0] == cid_j) & (pos[0] > j))
            dupC1 = dupC1 | ((cid[1] == cid_j) & (pos[1] > j))
            return clob0, clob1, dupP0, dupP1, dupC0, dupC1

        clob0, clob1, dupP0, dupP1, dupC0, dupC1 = lax.fori_loop(
            0, NGT, pair_body,
            (bfalse, bfalse, bfalse, bfalse, bfalse, bfalse))

        winner = [valid[0] & ~clob0, valid[1] & ~clob1]
        distinct = [valid[0] & ~dupP0, valid[1] & ~dupP1]
        dcell = [valid[0] & ~dupC0, valid[1] & ~dupC1]

        for ch in range(2):
            w = winner[ch]
            coord_acc += jnp.sum(jnp.where(w, se[ch], fzero))
            dob = psel4[ch] - ibest[ch]
            obj_acc += jnp.sum(jnp.where(w, dob * dob, fzero))
            confw_acc += jnp.sum(jnp.where(w, psel4[ch] * psel4[ch], fzero))
            gcls = plsc.load_gather(
                pred_v, [pbase + cid[ch] * F + (5 * NB) + lbls[ch]])
            b_acc += jnp.sum(jnp.where(distinct[ch], gcls, fzero))
            c_acc += jnp.sum(
                jnp.where(distinct[ch], jnp.ones((16,), jnp.float32), fzero))

        fb0 = cid[0] * F + pbase + (5 * NB)
        fb1 = cid[1] * F + pbase + (5 * NB)

        def css_body(l, carry):
            a0, a1 = carry
            for dl in range(4):
                g0 = plsc.load_gather(pred_v, [fb0 + l * 4 + dl])
                g1 = plsc.load_gather(pred_v, [fb1 + l * 4 + dl])
                a0 = a0 + g0 * g0
                a1 = a1 + g1 * g1
            return a0, a1

        cssv0, cssv1 = lax.fori_loop(0, C // 4, css_body, (fzero, fzero))
        clsA_acc += jnp.sum(jnp.where(dcell[0], cssv0, fzero))
        clsA_acc += jnp.sum(jnp.where(dcell[1], cssv1, fzero))

        for cc in range(4):
            cells = lane + 16 * cc
            cmask = cells < SS
            cellc = jnp.minimum(cells, SS - 1)
            v4 = plsc.load_gather(pred_v, [pbase + cellc * F + 4])
            v9 = plsc.load_gather(pred_v, [pbase + cellc * F + 9])
            confall_acc += jnp.sum(jnp.where(cmask, v4 * v4 + v9 * v9, fzero))
        return (coord_acc, obj_acc, confw_acc, confall_acc,
                clsA_acc, b_acc, c_acc)

    (coord_acc, obj_acc, confw_acc, confall_acc,
     clsA_acc, b_acc, c_acc) = lax.fori_loop(
        0, IPT, img_body,
        (coord_acc, obj_acc, confw_acc, confall_acc,
         clsA_acc, b_acc, c_acc))

    p = jnp.zeros((16,), jnp.float32)
    p = jnp.where(lane == 0, coord_acc * L_COORD, p)
    p = jnp.where(lane == 1, obj_acc, p)
    p = jnp.where(lane == 2, confw_acc, p)
    p = jnp.where(lane == 3, confall_acc, p)
    p = jnp.where(lane == 4, clsA_acc, p)
    p = jnp.where(lane == 5, b_acc, p)
    p = jnp.where(lane == 6, c_acc, p)
    part_v[...] = p
    pltpu.sync_copy(part_v, out_hbm.at[wid])


def kernel(preds, boxes, labels):
    preds_flat = preds.reshape(B * PB_W)
    box_flat = boxes.reshape(B * NGT * 4)
    lbl_flat = labels.reshape(B * NGT)
    parts = _sc_loss_kernel(preds_flat, box_flat, lbl_flat)
    coord_loss = jnp.sum(parts[:, 0])
    obj_loss = jnp.sum(parts[:, 1])
    noobj_loss = (jnp.sum(parts[:, 3]) - jnp.sum(parts[:, 2])) * L_NOOBJ
    cls_loss = (jnp.sum(parts[:, 4]) - 2.0 * jnp.sum(parts[:, 5])
                + jnp.sum(parts[:, 6]))
    total = coord_loss + obj_loss + noobj_loss + cls_loss
    return total, coord_loss, obj_loss, noobj_loss, cls_loss
